# Initial kernel scaffold; baseline (speedup 1.0000x reference)
import jax
import jax.numpy as jnp
from jax.experimental import pallas as pl


def kernel(feats, edge_index, W1, b1, W2, b2):
    raise NotImplementedError("write your pallas kernel here")


# speedup vs baseline: 4.3964x; 4.3964x over previous
import functools

import jax
import jax.numpy as jnp
from jax import lax
from jax.experimental import pallas as pl
from jax.experimental.pallas import tpu as pltpu
from jax.experimental.pallas import tpu_sc as plsc

N = 10000
NP = 10240
E = 320000
D = 128
NC = 2
NS = 16
NW = NC * NS
EPW = E // NW
C = 80
NCH = EPW // C
RPS = NP // NS
ZR = 128


HN = NP // 2


def _sc_degree_body(src_hbm, dst_hbm, hist_hbm, hbuf, idx_buf):
    c = lax.axis_index("c")
    s = lax.axis_index("s")
    wid = c * NS + s
    zeros16 = jnp.zeros((16,), jnp.float32)
    ones16 = jnp.ones((16,), jnp.float32)
    lanes = lax.iota(jnp.int32, 16)
    for t, ref in ((0, src_hbm), (1, dst_hbm)):
        pltpu.sync_copy(ref.at[pl.ds(wid * EPW, EPW)], idx_buf)
        for half in range(2):
            base = half * HN

            @pl.loop(0, HN, unroll=8)
            def _z(i):
                hbuf[pl.ds(i * 16, 16)] = zeros16

            @pl.loop(0, EPW // 16, unroll=4)
            def _h(i):
                v = idx_buf[pl.ds(i * 16, 16)]
                rel = v - base
                m = (rel >= 0) & (rel < HN)
                addr = rel + lanes * HN
                plsc.addupdate_scatter(hbuf, [addr], ones16, mask=m)

            off = (wid * 4 + t * 2 + half) * (HN * 16)
            pltpu.sync_copy(hbuf, hist_hbm.at[pl.ds(off, HN * 16)])


@functools.cache
def _build_sc_degrees():
  return pl.kernel(
    _sc_degree_body,
    out_type=jax.ShapeDtypeStruct((NW * 4 * HN * 16,), jnp.float32),
    mesh=plsc.VectorSubcoreMesh(core_axis_name="c", subcore_axis_name="s"),
    compiler_params=pltpu.CompilerParams(needs_layout_passes=False),
    scratch_types=[
        pltpu.VMEM((HN * 16,), jnp.float32),
        pltpu.VMEM((EPW,), jnp.int32),
    ],
  )


def _sc_scatter_body(g_hbm, src_hbm, dst_hbm, out_hbm,
                     acc, src_v, dst_v, rows_v, zbuf, sem):
    c = lax.axis_index("c")
    s = lax.axis_index("s")
    wid = c * NS + s
    zeros16 = jnp.zeros((16,), jnp.float32)

    @pl.loop(0, ZR)
    def _zb(i):
        for j in range(D // 16):
            zbuf[i, pl.ds(j * 16, 16)] = zeros16

    @pl.loop(0, RPS // ZR)
    def _za(k):
        pltpu.sync_copy(zbuf, acc.at[pl.ds(s * RPS + k * ZR, ZR)])

    plsc.subcore_barrier()

    @pl.loop(0, NCH)
    def _edges(i):
        e0 = wid * EPW + i * C
        pltpu.sync_copy(src_hbm.at[pl.ds(e0, C)], src_v)
        pltpu.sync_copy(dst_hbm.at[pl.ds(e0, C)], dst_v)
        pltpu.async_copy(g_hbm.at[src_v], rows_v, sem).wait()
        pltpu.sync_copy(rows_v, acc.at[dst_v], add=True)

    plsc.subcore_barrier()

    @pl.loop(0, RPS // ZR)
    def _wb(k):
        r0 = s * RPS + k * ZR
        pltpu.sync_copy(acc.at[pl.ds(r0, ZR)], out_hbm.at[c, pl.ds(r0, ZR)])


@functools.cache
def _build_sc_scatter():
  return pl.kernel(
    _sc_scatter_body,
    out_type=jax.ShapeDtypeStruct((NC, NP, D), jnp.float32),
    mesh=plsc.VectorSubcoreMesh(core_axis_name="c", subcore_axis_name="s"),
    compiler_params=pltpu.CompilerParams(needs_layout_passes=False),
    scratch_types=[
        pltpu.VMEM_SHARED((NP, D), jnp.float32),
        pltpu.VMEM((C,), jnp.int32),
        pltpu.VMEM((C,), jnp.int32),
        pltpu.VMEM((C, D), jnp.float32),
        pltpu.VMEM((ZR, D), jnp.float32),
        pltpu.SemaphoreType.DMA,
    ],
  )


_BM = 1000


def _tc0_body(hist_ref, deg_ref):
    deg_ref[...] = jnp.sum(hist_ref[...], axis=(0, 2))


def _tc0(hist4):
    return pl.pallas_call(
        _tc0_body,
        grid=(HN // 640,),
        in_specs=[pl.BlockSpec((NW, 4, 16, 640), lambda i: (0, 0, 0, i))],
        out_specs=pl.BlockSpec((4, 640), lambda i: (0, i)),
        out_shape=jax.ShapeDtypeStruct((4, HN), jnp.float32),
    )(hist4)


def _tc1_body(feats_ref, w_ref, degO_ref, degI_ref, g_ref, nO_ref, nI_ref):
    nO = lax.rsqrt(jnp.maximum(degO_ref[...], 1.0))
    nI = lax.rsqrt(jnp.maximum(degI_ref[...], 1.0))
    x = feats_ref[...] * nO
    g_ref[...] = jnp.dot(x, w_ref[...], preferred_element_type=jnp.float32)
    nO_ref[...] = nO
    nI_ref[...] = nI


def _tc1(feats, W1, degO, degI):
    return pl.pallas_call(
        _tc1_body,
        grid=(N // _BM,),
        in_specs=[
            pl.BlockSpec((_BM, D), lambda i: (i, 0)),
            pl.BlockSpec((D, D), lambda i: (0, 0)),
            pl.BlockSpec((_BM, 1), lambda i: (i, 0)),
            pl.BlockSpec((_BM, 1), lambda i: (i, 0)),
        ],
        out_specs=[
            pl.BlockSpec((_BM, D), lambda i: (i, 0)),
            pl.BlockSpec((_BM, 1), lambda i: (i, 0)),
            pl.BlockSpec((_BM, 1), lambda i: (i, 0)),
        ],
        out_shape=[jax.ShapeDtypeStruct((N, D), jnp.float32),
                   jax.ShapeDtypeStruct((N, 1), jnp.float32),
                   jax.ShapeDtypeStruct((N, 1), jnp.float32)],
    )(feats, W1, degO, degI)


def _tc2_body(agg_ref, nI_ref, b_ref, w_ref, nO_ref, g_ref):
    a = agg_ref[0] + agg_ref[1]
    h = jnp.maximum(a * nI_ref[...] + b_ref[...], 0.0)
    g_ref[...] = jnp.dot(h * nO_ref[...], w_ref[...],
                         preferred_element_type=jnp.float32)


def _tc2(agg, nI, b1, W2, nO):
    return pl.pallas_call(
        _tc2_body,
        grid=(N // _BM,),
        in_specs=[
            pl.BlockSpec((NC, _BM, D), lambda i: (0, i, 0)),
            pl.BlockSpec((_BM, 1), lambda i: (i, 0)),
            pl.BlockSpec((1, D), lambda i: (0, 0)),
            pl.BlockSpec((D, D), lambda i: (0, 0)),
            pl.BlockSpec((_BM, 1), lambda i: (i, 0)),
        ],
        out_specs=pl.BlockSpec((_BM, D), lambda i: (i, 0)),
        out_shape=jax.ShapeDtypeStruct((N, D), jnp.float32),
    )(agg, nI, b1, W2, nO)


def _tc3_body(agg_ref, nI_ref, b_ref, out_ref):
    a = agg_ref[0] + agg_ref[1]
    out_ref[...] = jnp.maximum(a * nI_ref[...] + b_ref[...], 0.0)


def _tc3(agg, nI, b2):
    return pl.pallas_call(
        _tc3_body,
        grid=(N // _BM,),
        in_specs=[
            pl.BlockSpec((NC, _BM, D), lambda i: (0, i, 0)),
            pl.BlockSpec((_BM, 1), lambda i: (i, 0)),
            pl.BlockSpec((1, D), lambda i: (0, 0)),
        ],
        out_specs=pl.BlockSpec((_BM, D), lambda i: (i, 0)),
        out_shape=jax.ShapeDtypeStruct((N, D), jnp.float32),
    )(agg, nI, b2)


def kernel(feats, edge_index, W1, b1, W2, b2):
    src = edge_index[0].astype(jnp.int32)
    dst = edge_index[1].astype(jnp.int32)
    feats = feats.astype(jnp.float32)
    hist = _build_sc_degrees()(src, dst)
    deg4 = _tc0(hist.reshape(NW, 4, 16, HN))
    degO = deg4[0:2].reshape(NP, 1)
    degI = deg4[2:4].reshape(NP, 1)
    g1, nO, nI = _tc1(feats, W1, degO[:N], degI[:N])
    agg1 = _build_sc_scatter()(g1, src, dst)
    g2 = _tc2(agg1, nI, b1.reshape(1, D), W2, nO)
    agg2 = _build_sc_scatter()(g2, src, dst)
    return _tc3(agg2, nI, b2.reshape(1, D))
